# baseline (speedup 1.0000x reference)
import jax
import jax.numpy as jnp
from jax.experimental import pallas as pl
from jax.experimental.pallas import tpu as pltpu

N = 10000
M_BLK = 400


def _gcn_body(supp_ref, b_ref, adj_ref, out_ref):
    acc = jnp.dot(adj_ref[...], supp_ref[...], preferred_element_type=jnp.float32)
    out_ref[...] = jnp.maximum(acc + b_ref[...], 0.0)


@jax.jit
def kernel(x, adj, W, b):
    n, nfeat = x.shape
    nhid = W.shape[1]
    support = jnp.dot(x, W)
    grid = (n // M_BLK,)
    return pl.pallas_call(
        _gcn_body,
        grid=grid,
        in_specs=[
            pl.BlockSpec((N, nhid), lambda m: (0, 0)),
            pl.BlockSpec((1, nhid), lambda m: (0, 0)),
            pl.BlockSpec((M_BLK, N), lambda m: (m, 0)),
        ],
        out_specs=pl.BlockSpec((M_BLK, nhid), lambda m: (m, 0)),
        out_shape=jax.ShapeDtypeStruct((n, nhid), jnp.float32),
    )(support, b.reshape(1, nhid), adj)


# speedup vs baseline: 1.0328x; 1.0328x over previous
import jax
import jax.numpy as jnp
from jax.experimental import pallas as pl
from jax.experimental.pallas import tpu as pltpu

N = 10000
NBUF = 5
M_CHUNK = 80
NCHUNKS = N // M_CHUNK


def _gcn_body(x_ref, w_ref, b_ref, adj_hbm, out_ref, supp_ref, *rest):
    bufs = rest[:NBUF]
    sems = rest[NBUF:]

    def start(chunk, slot):
        pltpu.make_async_copy(
            adj_hbm.at[pl.ds(chunk * M_CHUNK, M_CHUNK), :], bufs[slot], sems[slot]
        ).start()

    def wait(slot):
        pltpu.make_async_copy(
            adj_hbm.at[pl.ds(0, M_CHUNK), :], bufs[slot], sems[slot]
        ).wait()

    for s in range(NBUF):
        start(s, s)

    supp_ref[...] = jnp.dot(x_ref[...], w_ref[...], preferred_element_type=jnp.float32)
    supp = supp_ref[...]
    bias = b_ref[...]

    def outer(o, carry):
        for s in range(NBUF):
            c = o * NBUF + s
            wait(s)
            acc = jnp.dot(bufs[s][...], supp, preferred_element_type=jnp.float32)
            out_ref[pl.ds(c * M_CHUNK, M_CHUNK), :] = jnp.maximum(acc + bias, 0.0)

            @pl.when(c < NCHUNKS - NBUF)
            def _():
                start(c + NBUF, s)

        return carry

    jax.lax.fori_loop(0, NCHUNKS // NBUF, outer, 0)


@jax.jit
def kernel(x, adj, W, b):
    n, nfeat = x.shape
    nhid = W.shape[1]
    return pl.pallas_call(
        _gcn_body,
        in_specs=[
            pl.BlockSpec((n, nfeat), lambda: (0, 0)),
            pl.BlockSpec((nfeat, nhid), lambda: (0, 0)),
            pl.BlockSpec((1, nhid), lambda: (0, 0)),
            pl.BlockSpec(memory_space=pl.ANY),
        ],
        out_specs=pl.BlockSpec((n, nhid), lambda: (0, 0)),
        out_shape=jax.ShapeDtypeStruct((n, nhid), jnp.float32),
        scratch_shapes=(
            [pltpu.VMEM((N, nhid), jnp.float32)]
            + [pltpu.VMEM((M_CHUNK, N), jnp.float32) for _ in range(NBUF)]
            + [pltpu.SemaphoreType.DMA for _ in range(NBUF)]
        ),
    )(x, W, b.reshape(1, nhid), adj)
